# baseline (speedup 1.0000x reference)
import functools

import jax
import jax.numpy as jnp
from jax import lax
from jax.experimental import pallas as pl
from jax.experimental.pallas import tpu as pltpu
from jax.experimental.pallas import tpu_sc as plsc

B, T, D = 16, 512, 256
L = 2048
LANES = 16
NTOK_CH = T // LANES
NFRM_CH = L // LANES
ROWS = 64
NBUF = 5
ZROWS = ROWS
CHUNKS_A = tuple(range(0, L // ROWS, 2))
CHUNKS_B = tuple(range(1, L // ROWS, 2))
NSLOTS = max(len(CHUNKS_A), len(CHUNKS_B))


def _lr_body(table, dur, out, d_out, dur_v, d_v, z_v, gidx_v, rows_v, zero_v,
             *sems):
    gsem = sems[:NBUF]
    wsem = sems[NBUF:]
    h = lax.axis_index("c")
    b = (lax.axis_index("s") + 8 * h) % B

    with jax.named_scope("p0_load"):
        pltpu.sync_copy(dur.at[b], dur_v)

    with jax.named_scope("p1_zinit"):
        zeros16i = jnp.zeros((LANES,), jnp.int32)
        for i in range(NFRM_CH):
            z_v[pl.ds(i * LANES, LANES)] = zeros16i

    top = jnp.full((LANES,), LANES - 1, jnp.int32)

    def _bcast_last(v):
        return v.at[top].get(mode="promise_in_bounds")

    with jax.named_scope("p2_cumsum"):
        carry = jnp.zeros((LANES,), jnp.int32)
        ids0 = lax.broadcasted_iota(jnp.int32, (LANES,), 0)
        for i in range(NTOK_CH):
            dv = dur_v[pl.ds(i * LANES, LANES)]
            d16 = jnp.maximum(dv, 1)
            d_v[pl.ds(i * LANES, LANES)] = d16
            cum16 = plsc.cumsum(d16) + carry
            starts = cum16 - d16
            carry = _bcast_last(cum16)
            mask = starts < L
            starts_c = jnp.minimum(starts, L - 1)
            plsc.store_scatter(z_v, [starts_c], ids0 + (i * LANES), mask=mask)
        total = jnp.max(carry)

        @pl.when(h == b % 2)
        def _():
            pltpu.sync_copy(d_v, d_out.at[b])

    with jax.named_scope("p3_cummax"):
        mcarry = jnp.zeros((LANES,), jnp.int32)
        base_row = b * T
        for i in range(NFRM_CH):
            zc = z_v[pl.ds(i * LANES, LANES)]
            m = jnp.maximum(plsc.cummax(zc), mcarry)
            mcarry = _bcast_last(m)
            gidx_v[pl.ds(i * LANES, LANES)] = m + base_row

    zeros16f = jnp.zeros((LANES,), jnp.float32)

    def _zero_rows(ref, lo, hi):
        def body(r, _):
            for k in range(D // LANES):
                ref[r, pl.ds(k * LANES, LANES)] = zeros16f
            return 0
        lax.fori_loop(lo, hi, body, 0)


    def _slot(k):
        c0 = CHUNKS_A[k] if k < len(CHUNKS_A) else 0
        c1 = CHUNKS_B[k] if k < len(CHUNKS_B) else 0
        cid = jnp.where(h == 0, c0, c1)
        if k < len(CHUNKS_A) and k < len(CHUNKS_B):
            valid = (h == 0) | (h == 1)
        elif k < len(CHUNKS_A):
            valid = h == 0
        else:
            valid = h == 1
        start = cid * ROWS
        fb = pl.multiple_of(start, ROWS)
        live = jnp.clip(total - start, 0, ROWS)
        return fb, live, valid

    def _issue(k):
        buf = k % NBUF
        fb, live, valid = _slot(k)

        @pl.when(valid & (live > 0))
        def _():
            pltpu.async_copy(table.at[gidx_v.at[pl.ds(fb, ROWS)]],
                             rows_v.at[buf], gsem[buf])

    def _finish(k):
        buf = k % NBUF
        fb, live, valid = _slot(k)

        @pl.when(valid & (live > 0))
        def _():
            pltpu.make_async_copy(table.at[pl.ds(0, ROWS)], rows_v.at[buf],
                                  gsem[buf]).wait()

            @pl.when(live < ROWS)
            def _():
                _zero_rows(rows_v.at[buf], live, ROWS)

            pltpu.async_copy(rows_v.at[buf], out.at[b, pl.ds(fb, ROWS)],
                             wsem[buf])

        @pl.when(valid & (live == 0))
        def _():
            pltpu.async_copy(zero_v, out.at[b, pl.ds(fb, ZROWS)], wsem[buf])

    def _drain_write(k):
        buf = k % NBUF
        _, _, valid = _slot(k)

        @pl.when(valid)
        def _():
            pltpu.make_async_copy(table.at[pl.ds(0, ROWS)], rows_v.at[buf],
                                  wsem[buf]).wait()

    with jax.named_scope("p5_dma"):
        for k in range(NBUF):
            _issue(k)
        with jax.named_scope("p4_zbuf"):
            _zero_rows(zero_v, 0, ZROWS)
        for k in range(NSLOTS):
            _finish(k)
            if k + NBUF < NSLOTS:
                _drain_write(k)
                _issue(k + NBUF)
        for k in range(max(NSLOTS - NBUF, 0), NSLOTS):
            _drain_write(k)


def kernel(sequences, durations, max_mel_length):
    table = sequences.reshape(B * T, D)
    mesh = plsc.VectorSubcoreMesh(core_axis_name="c", subcore_axis_name="s")
    run = functools.partial(
        pl.kernel,
        mesh=mesh,
        compiler_params=pltpu.CompilerParams(needs_layout_passes=False),
        out_type=(jax.ShapeDtypeStruct((B, L, D), jnp.float32),
                  jax.ShapeDtypeStruct((B, T), jnp.int32)),
        scratch_types=[
            pltpu.VMEM((T,), jnp.int32),
            pltpu.VMEM((T,), jnp.int32),
            pltpu.VMEM((L,), jnp.int32),
            pltpu.VMEM((L,), jnp.int32),
            pltpu.VMEM((NBUF, ROWS, D), jnp.float32),
            pltpu.VMEM((ZROWS, D), jnp.float32),
        ] + [pltpu.SemaphoreType.DMA] * (2 * NBUF),
    )(_lr_body)
    out, d = run(table, durations)
    return out, d


# speedup vs baseline: 1.0108x; 1.0108x over previous
import functools

import jax
import jax.numpy as jnp
from jax import lax
from jax.experimental import pallas as pl
from jax.experimental.pallas import tpu as pltpu
from jax.experimental.pallas import tpu_sc as plsc

B, T, D = 16, 512, 256
L = 2048
LANES = 16
NTOK_CH = T // LANES
NFRM_CH = L // LANES
ROWS = 64
NBUF = 6
ZROWS = ROWS
CHUNKS_A = tuple(range(0, L // ROWS, 2))
CHUNKS_B = tuple(range(1, L // ROWS, 2))
NSLOTS = max(len(CHUNKS_A), len(CHUNKS_B))


def _lr_body(table, dur, out, d_out, dur_v, d_v, z_v, gidx_v, rows_v, zero_v,
             *sems):
    gsem = sems[:NBUF]
    wsem = sems[NBUF:]
    h = lax.axis_index("c")
    b = (lax.axis_index("s") + 8 * h) % B

    with jax.named_scope("p0_load"):
        pltpu.sync_copy(dur.at[b], dur_v)

    with jax.named_scope("p1_zinit"):
        zeros16i = jnp.zeros((LANES,), jnp.int32)
        for i in range(NFRM_CH):
            z_v[pl.ds(i * LANES, LANES)] = zeros16i

    top = jnp.full((LANES,), LANES - 1, jnp.int32)

    def _bcast_last(v):
        return v.at[top].get(mode="promise_in_bounds")

    with jax.named_scope("p2_cumsum"):
        carry = jnp.zeros((LANES,), jnp.int32)
        ids0 = lax.broadcasted_iota(jnp.int32, (LANES,), 0)
        for i in range(NTOK_CH):
            dv = dur_v[pl.ds(i * LANES, LANES)]
            d16 = jnp.maximum(dv, 1)
            d_v[pl.ds(i * LANES, LANES)] = d16
            cum16 = plsc.cumsum(d16) + carry
            starts = cum16 - d16
            carry = _bcast_last(cum16)
            mask = starts < L
            starts_c = jnp.minimum(starts, L - 1)
            plsc.store_scatter(z_v, [starts_c], ids0 + (i * LANES), mask=mask)
        total = jnp.max(carry)

        @pl.when(h == b % 2)
        def _():
            pltpu.sync_copy(d_v, d_out.at[b])

    with jax.named_scope("p3_cummax"):
        mcarry = jnp.zeros((LANES,), jnp.int32)
        base_row = b * T
        for i in range(NFRM_CH):
            zc = z_v[pl.ds(i * LANES, LANES)]
            m = jnp.maximum(plsc.cummax(zc), mcarry)
            mcarry = _bcast_last(m)
            gidx_v[pl.ds(i * LANES, LANES)] = m + base_row

    zeros16f = jnp.zeros((LANES,), jnp.float32)

    def _zero_rows(ref, lo, hi):
        def body(r, _):
            for k in range(D // LANES):
                ref[r, pl.ds(k * LANES, LANES)] = zeros16f
            return 0
        lax.fori_loop(lo, hi, body, 0)


    def _slot(k):
        c0 = CHUNKS_A[k] if k < len(CHUNKS_A) else 0
        c1 = CHUNKS_B[k] if k < len(CHUNKS_B) else 0
        cid = jnp.where(h == 0, c0, c1)
        if k < len(CHUNKS_A) and k < len(CHUNKS_B):
            valid = (h == 0) | (h == 1)
        elif k < len(CHUNKS_A):
            valid = h == 0
        else:
            valid = h == 1
        start = cid * ROWS
        fb = pl.multiple_of(start, ROWS)
        live = jnp.clip(total - start, 0, ROWS)
        return fb, live, valid

    def _issue(k):
        buf = k % NBUF
        fb, live, valid = _slot(k)

        @pl.when(valid & (live > 0))
        def _():
            pltpu.async_copy(table.at[gidx_v.at[pl.ds(fb, ROWS)]],
                             rows_v.at[buf], gsem[buf])

    def _finish(k):
        buf = k % NBUF
        fb, live, valid = _slot(k)

        @pl.when(valid & (live > 0))
        def _():
            pltpu.make_async_copy(table.at[pl.ds(0, ROWS)], rows_v.at[buf],
                                  gsem[buf]).wait()

            @pl.when(live < ROWS)
            def _():
                _zero_rows(rows_v.at[buf], live, ROWS)

            pltpu.async_copy(rows_v.at[buf], out.at[b, pl.ds(fb, ROWS)],
                             wsem[buf])

        @pl.when(valid & (live == 0))
        def _():
            pltpu.async_copy(zero_v, out.at[b, pl.ds(fb, ZROWS)], wsem[buf])

    def _drain_write(k):
        buf = k % NBUF
        _, _, valid = _slot(k)

        @pl.when(valid)
        def _():
            pltpu.make_async_copy(table.at[pl.ds(0, ROWS)], rows_v.at[buf],
                                  wsem[buf]).wait()

    with jax.named_scope("p5_dma"):
        for k in range(NBUF):
            _issue(k)
        with jax.named_scope("p4_zbuf"):
            _zero_rows(zero_v, 0, ZROWS)
        for k in range(NSLOTS):
            _finish(k)
            if k + NBUF < NSLOTS:
                _drain_write(k)
                _issue(k + NBUF)
        for k in range(max(NSLOTS - NBUF, 0), NSLOTS):
            _drain_write(k)


def kernel(sequences, durations, max_mel_length):
    table = sequences.reshape(B * T, D)
    mesh = plsc.VectorSubcoreMesh(core_axis_name="c", subcore_axis_name="s")
    run = functools.partial(
        pl.kernel,
        mesh=mesh,
        compiler_params=pltpu.CompilerParams(needs_layout_passes=False),
        out_type=(jax.ShapeDtypeStruct((B, L, D), jnp.float32),
                  jax.ShapeDtypeStruct((B, T), jnp.int32)),
        scratch_types=[
            pltpu.VMEM((T,), jnp.int32),
            pltpu.VMEM((T,), jnp.int32),
            pltpu.VMEM((L,), jnp.int32),
            pltpu.VMEM((L,), jnp.int32),
            pltpu.VMEM((NBUF, ROWS, D), jnp.float32),
            pltpu.VMEM((ZROWS, D), jnp.float32),
        ] + [pltpu.SemaphoreType.DMA] * (2 * NBUF),
    )(_lr_body)
    out, d = run(table, durations)
    return out, d


# speedup vs baseline: 1.0197x; 1.0089x over previous
import functools

import jax
import jax.numpy as jnp
from jax import lax
from jax.experimental import pallas as pl
from jax.experimental.pallas import tpu as pltpu
from jax.experimental.pallas import tpu_sc as plsc

B, T, D = 16, 512, 256
L = 2048
LANES = 16
NTOK_CH = T // LANES
NFRM_CH = L // LANES
ROWS = 32
NBUF = 10
ZROWS = ROWS
CHUNKS_A = tuple(range(0, L // ROWS, 2))
CHUNKS_B = tuple(range(1, L // ROWS, 2))
NSLOTS = max(len(CHUNKS_A), len(CHUNKS_B))


def _lr_body(table, dur, out, d_out, dur_v, d_v, z_v, gidx_v, rows_v, zero_v,
             *sems):
    gsem = sems[:NBUF]
    wsem = sems[NBUF:]
    h = lax.axis_index("c")
    b = (lax.axis_index("s") + 8 * h) % B

    with jax.named_scope("p0_load"):
        pltpu.sync_copy(dur.at[b], dur_v)

    with jax.named_scope("p1_zinit"):
        zeros16i = jnp.zeros((LANES,), jnp.int32)
        for i in range(NFRM_CH):
            z_v[pl.ds(i * LANES, LANES)] = zeros16i

    top = jnp.full((LANES,), LANES - 1, jnp.int32)

    def _bcast_last(v):
        return v.at[top].get(mode="promise_in_bounds")

    with jax.named_scope("p2_cumsum"):
        carry = jnp.zeros((LANES,), jnp.int32)
        ids0 = lax.broadcasted_iota(jnp.int32, (LANES,), 0)
        for i in range(NTOK_CH):
            dv = dur_v[pl.ds(i * LANES, LANES)]
            d16 = jnp.maximum(dv, 1)
            d_v[pl.ds(i * LANES, LANES)] = d16
            cum16 = plsc.cumsum(d16) + carry
            starts = cum16 - d16
            carry = _bcast_last(cum16)
            mask = starts < L
            starts_c = jnp.minimum(starts, L - 1)
            plsc.store_scatter(z_v, [starts_c], ids0 + (i * LANES), mask=mask)
        total = jnp.max(carry)

        @pl.when(h == b % 2)
        def _():
            pltpu.sync_copy(d_v, d_out.at[b])

    with jax.named_scope("p3_cummax"):
        mcarry = jnp.zeros((LANES,), jnp.int32)
        base_row = b * T
        for i in range(NFRM_CH):
            zc = z_v[pl.ds(i * LANES, LANES)]
            m = jnp.maximum(plsc.cummax(zc), mcarry)
            mcarry = _bcast_last(m)
            gidx_v[pl.ds(i * LANES, LANES)] = m + base_row

    zeros16f = jnp.zeros((LANES,), jnp.float32)

    def _zero_rows(ref, lo, hi):
        def body(r, _):
            for k in range(D // LANES):
                ref[r, pl.ds(k * LANES, LANES)] = zeros16f
            return 0
        lax.fori_loop(lo, hi, body, 0)


    def _slot(k):
        c0 = CHUNKS_A[k] if k < len(CHUNKS_A) else 0
        c1 = CHUNKS_B[k] if k < len(CHUNKS_B) else 0
        cid = jnp.where(h == 0, c0, c1)
        if k < len(CHUNKS_A) and k < len(CHUNKS_B):
            valid = (h == 0) | (h == 1)
        elif k < len(CHUNKS_A):
            valid = h == 0
        else:
            valid = h == 1
        start = cid * ROWS
        fb = pl.multiple_of(start, ROWS)
        live = jnp.clip(total - start, 0, ROWS)
        return fb, live, valid

    def _issue(k):
        buf = k % NBUF
        fb, live, valid = _slot(k)

        @pl.when(valid & (live > 0))
        def _():
            pltpu.async_copy(table.at[gidx_v.at[pl.ds(fb, ROWS)]],
                             rows_v.at[buf], gsem[buf])

    def _finish(k):
        buf = k % NBUF
        fb, live, valid = _slot(k)

        @pl.when(valid & (live > 0))
        def _():
            pltpu.make_async_copy(table.at[pl.ds(0, ROWS)], rows_v.at[buf],
                                  gsem[buf]).wait()

            @pl.when(live < ROWS)
            def _():
                _zero_rows(rows_v.at[buf], live, ROWS)

            pltpu.async_copy(rows_v.at[buf], out.at[b, pl.ds(fb, ROWS)],
                             wsem[buf])

        @pl.when(valid & (live == 0))
        def _():
            pltpu.async_copy(zero_v, out.at[b, pl.ds(fb, ZROWS)], wsem[buf])

    def _drain_write(k):
        buf = k % NBUF
        _, _, valid = _slot(k)

        @pl.when(valid)
        def _():
            pltpu.make_async_copy(table.at[pl.ds(0, ROWS)], rows_v.at[buf],
                                  wsem[buf]).wait()

    with jax.named_scope("p5_dma"):
        for k in range(NBUF):
            _issue(k)
        with jax.named_scope("p4_zbuf"):
            _zero_rows(zero_v, 0, ZROWS)
        for k in range(NSLOTS):
            _finish(k)
            if k + NBUF < NSLOTS:
                _drain_write(k)
                _issue(k + NBUF)
        for k in range(max(NSLOTS - NBUF, 0), NSLOTS):
            _drain_write(k)


def kernel(sequences, durations, max_mel_length):
    table = sequences.reshape(B * T, D)
    mesh = plsc.VectorSubcoreMesh(core_axis_name="c", subcore_axis_name="s")
    run = functools.partial(
        pl.kernel,
        mesh=mesh,
        compiler_params=pltpu.CompilerParams(needs_layout_passes=False),
        out_type=(jax.ShapeDtypeStruct((B, L, D), jnp.float32),
                  jax.ShapeDtypeStruct((B, T), jnp.int32)),
        scratch_types=[
            pltpu.VMEM((T,), jnp.int32),
            pltpu.VMEM((T,), jnp.int32),
            pltpu.VMEM((L,), jnp.int32),
            pltpu.VMEM((L,), jnp.int32),
            pltpu.VMEM((NBUF, ROWS, D), jnp.float32),
            pltpu.VMEM((ZROWS, D), jnp.float32),
        ] + [pltpu.SemaphoreType.DMA] * (2 * NBUF),
    )(_lr_body)
    out, d = run(table, durations)
    return out, d


# speedup vs baseline: 1.0251x; 1.0053x over previous
import functools

import jax
import jax.numpy as jnp
from jax import lax
from jax.experimental import pallas as pl
from jax.experimental.pallas import tpu as pltpu
from jax.experimental.pallas import tpu_sc as plsc

B, T, D = 16, 512, 256
L = 2048
LANES = 16
NTOK_CH = T // LANES
NFRM_CH = L // LANES
ROWS = 32
NBUF = 10
ZROWS = ROWS
CHUNKS_A = tuple(range(0, L // ROWS, 2))
CHUNKS_B = tuple(range(1, L // ROWS, 2))
NSLOTS = max(len(CHUNKS_A), len(CHUNKS_B))


def _lr_body(table, dur, out, d_out, dur_v, d_v, z_v, gidx_v, rows_v, zero_v,
             *sems):
    gsem = sems[:NBUF]
    wsem = sems[NBUF:]
    h = lax.axis_index("c")
    b = (lax.axis_index("s") + 8 * h) % B

    with jax.named_scope("p0_load"):
        pltpu.sync_copy(dur.at[b], dur_v)

    with jax.named_scope("p1_zinit"):
        zeros16i = jnp.zeros((LANES,), jnp.int32)
        for i in range(NFRM_CH):
            z_v[pl.ds(i * LANES, LANES)] = zeros16i

    top = jnp.full((LANES,), LANES - 1, jnp.int32)

    def _bcast_last(v):
        return v.at[top].get(mode="promise_in_bounds")

    with jax.named_scope("p2_cumsum"):
        carry = jnp.zeros((LANES,), jnp.int32)
        ids0 = lax.broadcasted_iota(jnp.int32, (LANES,), 0)
        for i in range(NTOK_CH):
            dv = dur_v[pl.ds(i * LANES, LANES)]
            d16 = jnp.maximum(dv, 1)
            d_v[pl.ds(i * LANES, LANES)] = d16
            cum16 = plsc.cumsum(d16) + carry
            starts = cum16 - d16
            carry = _bcast_last(cum16)
            mask = starts < L
            starts_c = jnp.minimum(starts, L - 1)
            plsc.store_scatter(z_v, [starts_c], ids0 + (i * LANES), mask=mask)
        total = jnp.max(carry)

        @pl.when(h == b % 2)
        def _():
            pltpu.sync_copy(d_v, d_out.at[b])

    zeros16f = jnp.zeros((LANES,), jnp.float32)

    def _zero_rows(ref, lo, hi):
        def body(r, _):
            for k in range(D // LANES):
                ref[r, pl.ds(k * LANES, LANES)] = zeros16f
            return 0
        lax.fori_loop(lo, hi, body, 0)


    def _slot(k):
        c0 = CHUNKS_A[k] if k < len(CHUNKS_A) else 0
        c1 = CHUNKS_B[k] if k < len(CHUNKS_B) else 0
        cid = jnp.where(h == 0, c0, c1)
        if k < len(CHUNKS_A) and k < len(CHUNKS_B):
            valid = (h == 0) | (h == 1)
        elif k < len(CHUNKS_A):
            valid = h == 0
        else:
            valid = h == 1
        start = cid * ROWS
        fb = pl.multiple_of(start, ROWS)
        live = jnp.clip(total - start, 0, ROWS)
        return fb, live, valid

    def _issue(k):
        buf = k % NBUF
        fb, live, valid = _slot(k)

        @pl.when(valid & (live > 0))
        def _():
            pltpu.async_copy(table.at[gidx_v.at[pl.ds(fb, ROWS)]],
                             rows_v.at[buf], gsem[buf])

    def _finish(k):
        buf = k % NBUF
        fb, live, valid = _slot(k)

        @pl.when(valid & (live > 0))
        def _():
            pltpu.make_async_copy(table.at[pl.ds(0, ROWS)], rows_v.at[buf],
                                  gsem[buf]).wait()

            @pl.when(live < ROWS)
            def _():
                _zero_rows(rows_v.at[buf], live, ROWS)

            pltpu.async_copy(rows_v.at[buf], out.at[b, pl.ds(fb, ROWS)],
                             wsem[buf])

        @pl.when(valid & (live == 0))
        def _():
            pltpu.async_copy(zero_v, out.at[b, pl.ds(fb, ZROWS)], wsem[buf])

    def _drain_write(k):
        buf = k % NBUF
        _, _, valid = _slot(k)

        @pl.when(valid)
        def _():
            pltpu.make_async_copy(table.at[pl.ds(0, ROWS)], rows_v.at[buf],
                                  wsem[buf]).wait()

    with jax.named_scope("p3_cummax"):
        mcarry = jnp.zeros((LANES,), jnp.int32)
        base_row = b * T
        for i in range(NFRM_CH):
            zc = z_v[pl.ds(i * LANES, LANES)]
            m = jnp.maximum(plsc.cummax(zc), mcarry)
            mcarry = _bcast_last(m)
            gidx_v[pl.ds(i * LANES, LANES)] = m + base_row
            if i % 4 == 1 and (i - 1) // 4 < NBUF:
                @pl.when(h == 0)
                def _(k=(i - 1) // 4):
                    _issue(k)
            if i % 4 == 3 and (i - 3) // 4 < NBUF:
                @pl.when(h == 1)
                def _(k=(i - 3) // 4):
                    _issue(k)

    with jax.named_scope("p5_dma"):
        with jax.named_scope("p4_zbuf"):
            _zero_rows(zero_v, 0, ZROWS)
        for k in range(NSLOTS):
            _finish(k)
            if k + NBUF < NSLOTS:
                _drain_write(k)
                _issue(k + NBUF)
        for k in range(max(NSLOTS - NBUF, 0), NSLOTS):
            _drain_write(k)


def kernel(sequences, durations, max_mel_length):
    table = sequences.reshape(B * T, D)
    mesh = plsc.VectorSubcoreMesh(core_axis_name="c", subcore_axis_name="s")
    run = functools.partial(
        pl.kernel,
        mesh=mesh,
        compiler_params=pltpu.CompilerParams(needs_layout_passes=False),
        out_type=(jax.ShapeDtypeStruct((B, L, D), jnp.float32),
                  jax.ShapeDtypeStruct((B, T), jnp.int32)),
        scratch_types=[
            pltpu.VMEM((T,), jnp.int32),
            pltpu.VMEM((T,), jnp.int32),
            pltpu.VMEM((L,), jnp.int32),
            pltpu.VMEM((L,), jnp.int32),
            pltpu.VMEM((NBUF, ROWS, D), jnp.float32),
            pltpu.VMEM((ZROWS, D), jnp.float32),
        ] + [pltpu.SemaphoreType.DMA] * (2 * NBUF),
    )(_lr_body)
    out, d = run(table, durations)
    return out, d


# speedup vs baseline: 1.0269x; 1.0018x over previous
import functools

import jax
import jax.numpy as jnp
from jax import lax
from jax.experimental import pallas as pl
from jax.experimental.pallas import tpu as pltpu
from jax.experimental.pallas import tpu_sc as plsc

B, T, D = 16, 512, 256
L = 2048
LANES = 16
NTOK_CH = T // LANES
NFRM_CH = L // LANES
ROWS = 32
NBUF = 12
ZROWS = ROWS
CHUNKS_A = tuple(range(0, L // ROWS, 2))
CHUNKS_B = tuple(range(1, L // ROWS, 2))
NSLOTS = max(len(CHUNKS_A), len(CHUNKS_B))


def _lr_body(table, dur, out, d_out, dur_v, d_v, z_v, gidx_v, rows_v, zero_v,
             *sems):
    gsem = sems[:NBUF]
    wsem = sems[NBUF:]
    h = lax.axis_index("c")
    b = (lax.axis_index("s") + 8 * h) % B

    with jax.named_scope("p0_load"):
        pltpu.sync_copy(dur.at[b], dur_v)

    with jax.named_scope("p1_zinit"):
        zeros16i = jnp.zeros((LANES,), jnp.int32)
        for i in range(NFRM_CH):
            z_v[pl.ds(i * LANES, LANES)] = zeros16i

    top = jnp.full((LANES,), LANES - 1, jnp.int32)

    def _bcast_last(v):
        return v.at[top].get(mode="promise_in_bounds")

    with jax.named_scope("p2_cumsum"):
        carry = jnp.zeros((LANES,), jnp.int32)
        ids0 = lax.broadcasted_iota(jnp.int32, (LANES,), 0)
        for i in range(NTOK_CH):
            dv = dur_v[pl.ds(i * LANES, LANES)]
            d16 = jnp.maximum(dv, 1)
            d_v[pl.ds(i * LANES, LANES)] = d16
            cum16 = plsc.cumsum(d16) + carry
            starts = cum16 - d16
            carry = _bcast_last(cum16)
            mask = starts < L
            starts_c = jnp.minimum(starts, L - 1)
            plsc.store_scatter(z_v, [starts_c], ids0 + (i * LANES), mask=mask)
        total = jnp.max(carry)

        @pl.when(h == b % 2)
        def _():
            pltpu.sync_copy(d_v, d_out.at[b])

    zeros16f = jnp.zeros((LANES,), jnp.float32)

    def _zero_rows(ref, lo, hi):
        def body(r, _):
            for k in range(D // LANES):
                ref[r, pl.ds(k * LANES, LANES)] = zeros16f
            return 0
        lax.fori_loop(lo, hi, body, 0)


    def _slot(k):
        c0 = CHUNKS_A[k] if k < len(CHUNKS_A) else 0
        c1 = CHUNKS_B[k] if k < len(CHUNKS_B) else 0
        cid = jnp.where(h == 0, c0, c1)
        if k < len(CHUNKS_A) and k < len(CHUNKS_B):
            valid = (h == 0) | (h == 1)
        elif k < len(CHUNKS_A):
            valid = h == 0
        else:
            valid = h == 1
        start = cid * ROWS
        fb = pl.multiple_of(start, ROWS)
        live = jnp.clip(total - start, 0, ROWS)
        return fb, live, valid

    def _issue(k):
        buf = k % NBUF
        fb, live, valid = _slot(k)

        @pl.when(valid & (live > 0))
        def _():
            pltpu.async_copy(table.at[gidx_v.at[pl.ds(fb, ROWS)]],
                             rows_v.at[buf], gsem[buf])

    def _finish(k):
        buf = k % NBUF
        fb, live, valid = _slot(k)

        @pl.when(valid & (live > 0))
        def _():
            pltpu.make_async_copy(table.at[pl.ds(0, ROWS)], rows_v.at[buf],
                                  gsem[buf]).wait()

            @pl.when(live < ROWS)
            def _():
                _zero_rows(rows_v.at[buf], live, ROWS)

            pltpu.async_copy(rows_v.at[buf], out.at[b, pl.ds(fb, ROWS)],
                             wsem[buf])

        @pl.when(valid & (live == 0))
        def _():
            pltpu.async_copy(zero_v, out.at[b, pl.ds(fb, ZROWS)], wsem[buf])

    def _drain_write(k):
        buf = k % NBUF
        _, _, valid = _slot(k)

        @pl.when(valid)
        def _():
            pltpu.make_async_copy(table.at[pl.ds(0, ROWS)], rows_v.at[buf],
                                  wsem[buf]).wait()

    with jax.named_scope("p3_cummax"):
        mcarry = jnp.zeros((LANES,), jnp.int32)
        base_row = b * T
        for i in range(NFRM_CH):
            zc = z_v[pl.ds(i * LANES, LANES)]
            m = jnp.maximum(plsc.cummax(zc), mcarry)
            mcarry = _bcast_last(m)
            gidx_v[pl.ds(i * LANES, LANES)] = m + base_row
            if i % 4 == 1 and (i - 1) // 4 < NBUF:
                @pl.when(h == 0)
                def _(k=(i - 1) // 4):
                    _issue(k)
            if i % 4 == 3 and (i - 3) // 4 < NBUF:
                @pl.when(h == 1)
                def _(k=(i - 3) // 4):
                    _issue(k)

    with jax.named_scope("p5_dma"):
        with jax.named_scope("p4_zbuf"):
            _zero_rows(zero_v, 0, ZROWS)
        for k in range(NSLOTS):
            _finish(k)
            if k + NBUF < NSLOTS:
                _drain_write(k)
                _issue(k + NBUF)
        for k in range(max(NSLOTS - NBUF, 0), NSLOTS):
            _drain_write(k)


def kernel(sequences, durations, max_mel_length):
    table = sequences.reshape(B * T, D)
    mesh = plsc.VectorSubcoreMesh(core_axis_name="c", subcore_axis_name="s")
    run = functools.partial(
        pl.kernel,
        mesh=mesh,
        compiler_params=pltpu.CompilerParams(needs_layout_passes=False),
        out_type=(jax.ShapeDtypeStruct((B, L, D), jnp.float32),
                  jax.ShapeDtypeStruct((B, T), jnp.int32)),
        scratch_types=[
            pltpu.VMEM((T,), jnp.int32),
            pltpu.VMEM((T,), jnp.int32),
            pltpu.VMEM((L,), jnp.int32),
            pltpu.VMEM((L,), jnp.int32),
            pltpu.VMEM((NBUF, ROWS, D), jnp.float32),
            pltpu.VMEM((ZROWS, D), jnp.float32),
        ] + [pltpu.SemaphoreType.DMA] * (2 * NBUF),
    )(_lr_body)
    out, d = run(table, durations)
    return out, d


# speedup vs baseline: 1.0279x; 1.0009x over previous
import functools

import jax
import jax.numpy as jnp
from jax import lax
from jax.experimental import pallas as pl
from jax.experimental.pallas import tpu as pltpu
from jax.experimental.pallas import tpu_sc as plsc

B, T, D = 16, 512, 256
L = 2048
LANES = 16
NTOK_CH = T // LANES
NFRM_CH = L // LANES
ROWS = 32
NBUF = 12
ZROWS = ROWS
CHUNKS_A = tuple(range(0, L // ROWS, 2))
CHUNKS_B = tuple(range(1, L // ROWS, 2))
NSLOTS = max(len(CHUNKS_A), len(CHUNKS_B))


def _lr_body(table, dur, out, d_out, dur_v, d_v, z_v, gidx_v, rows_v, zero_v,
             *sems):
    gsem = sems[:NBUF]
    wsem = sems[NBUF:]
    h = lax.axis_index("c")
    b = (lax.axis_index("s") + 8 * h) % B

    dur_cp = pltpu.async_copy(dur.at[b], dur_v, gsem[0])

    zeros16i = jnp.zeros((LANES,), jnp.int32)
    for i in range(NFRM_CH):
        z_v[pl.ds(i * LANES, LANES)] = zeros16i
    dur_cp.wait()

    top = jnp.full((LANES,), LANES - 1, jnp.int32)

    def _bcast_last(v):
        return v.at[top].get(mode="promise_in_bounds")

    carry = jnp.zeros((LANES,), jnp.int32)
    ids0 = lax.broadcasted_iota(jnp.int32, (LANES,), 0)
    for i in range(NTOK_CH):
        dv = dur_v[pl.ds(i * LANES, LANES)]
        d16 = jnp.maximum(dv, 1)
        d_v[pl.ds(i * LANES, LANES)] = d16
        cum16 = plsc.cumsum(d16) + carry
        starts = cum16 - d16
        carry = _bcast_last(cum16)
        mask = starts < L
        starts_c = jnp.minimum(starts, L - 1)
        plsc.store_scatter(z_v, [starts_c], ids0 + (i * LANES), mask=mask)
    total = jnp.max(carry)

    @pl.when(h == b % 2)
    def _():
        pltpu.sync_copy(d_v, d_out.at[b])

    zeros16f = jnp.zeros((LANES,), jnp.float32)

    def _zero_rows(ref, lo, hi):
        def body(r, _):
            for k in range(D // LANES):
                ref[r, pl.ds(k * LANES, LANES)] = zeros16f
            return 0
        lax.fori_loop(lo, hi, body, 0)


    def _slot(k):
        c0 = CHUNKS_A[k] if k < len(CHUNKS_A) else 0
        c1 = CHUNKS_B[k] if k < len(CHUNKS_B) else 0
        cid = jnp.where(h == 0, c0, c1)
        if k < len(CHUNKS_A) and k < len(CHUNKS_B):
            valid = (h == 0) | (h == 1)
        elif k < len(CHUNKS_A):
            valid = h == 0
        else:
            valid = h == 1
        start = cid * ROWS
        fb = pl.multiple_of(start, ROWS)
        live = jnp.clip(total - start, 0, ROWS)
        return fb, live, valid

    def _issue(k):
        buf = k % NBUF
        fb, live, valid = _slot(k)

        @pl.when(valid & (live > 0))
        def _():
            pltpu.async_copy(table.at[gidx_v.at[pl.ds(fb, ROWS)]],
                             rows_v.at[buf], gsem[buf])

    def _finish(k):
        buf = k % NBUF
        fb, live, valid = _slot(k)

        @pl.when(valid & (live > 0))
        def _():
            pltpu.make_async_copy(table.at[pl.ds(0, ROWS)], rows_v.at[buf],
                                  gsem[buf]).wait()

            @pl.when(live < ROWS)
            def _():
                _zero_rows(rows_v.at[buf], live, ROWS)

            pltpu.async_copy(rows_v.at[buf], out.at[b, pl.ds(fb, ROWS)],
                             wsem[buf])

        @pl.when(valid & (live == 0))
        def _():
            pltpu.async_copy(zero_v, out.at[b, pl.ds(fb, ZROWS)], wsem[buf])

    def _drain_write(k):
        buf = k % NBUF
        _, _, valid = _slot(k)

        @pl.when(valid)
        def _():
            pltpu.make_async_copy(table.at[pl.ds(0, ROWS)], rows_v.at[buf],
                                  wsem[buf]).wait()

    mcarry = jnp.zeros((LANES,), jnp.int32)
    base_row = b * T
    for i in range(NFRM_CH):
        zc = z_v[pl.ds(i * LANES, LANES)]
        m = jnp.maximum(plsc.cummax(zc), mcarry)
        mcarry = _bcast_last(m)
        gidx_v[pl.ds(i * LANES, LANES)] = m + base_row
        if i % 4 == 1 and (i - 1) // 4 < NBUF:
            @pl.when(h == 0)
            def _(k=(i - 1) // 4):
                _issue(k)
        if i % 4 == 3 and (i - 3) // 4 < NBUF:
            @pl.when(h == 1)
            def _(k=(i - 3) // 4):
                _issue(k)

    _zero_rows(zero_v, 0, ZROWS)
    for k in range(NSLOTS):
        _finish(k)
        if k + NBUF < NSLOTS:
            _drain_write(k)
            _issue(k + NBUF)
    for k in range(max(NSLOTS - NBUF, 0), NSLOTS):
        _drain_write(k)


def kernel(sequences, durations, max_mel_length):
    table = sequences.reshape(B * T, D)
    mesh = plsc.VectorSubcoreMesh(core_axis_name="c", subcore_axis_name="s")
    run = functools.partial(
        pl.kernel,
        mesh=mesh,
        compiler_params=pltpu.CompilerParams(needs_layout_passes=False),
        out_type=(jax.ShapeDtypeStruct((B, L, D), jnp.float32),
                  jax.ShapeDtypeStruct((B, T), jnp.int32)),
        scratch_types=[
            pltpu.VMEM((T,), jnp.int32),
            pltpu.VMEM((T,), jnp.int32),
            pltpu.VMEM((L,), jnp.int32),
            pltpu.VMEM((L,), jnp.int32),
            pltpu.VMEM((NBUF, ROWS, D), jnp.float32),
            pltpu.VMEM((ZROWS, D), jnp.float32),
        ] + [pltpu.SemaphoreType.DMA] * (2 * NBUF),
    )(_lr_body)
    out, d = run(table, durations)
    return out, d


# speedup vs baseline: 1.0297x; 1.0018x over previous
import functools

import jax
import jax.numpy as jnp
from jax import lax
from jax.experimental import pallas as pl
from jax.experimental.pallas import tpu as pltpu
from jax.experimental.pallas import tpu_sc as plsc

B, T, D = 16, 512, 256
L = 2048
LANES = 16
NTOK_CH = T // LANES
NFRM_CH = L // LANES
ROWS = 32
NBUF = 14
ZROWS = ROWS
CHUNKS_A = tuple(range(0, L // ROWS, 2))
CHUNKS_B = tuple(range(1, L // ROWS, 2))
NSLOTS = max(len(CHUNKS_A), len(CHUNKS_B))


def _lr_body(table, dur, out, d_out, dur_v, d_v, z_v, gidx_v, rows_v, zero_v,
             *sems):
    gsem = sems[:NBUF]
    wsem = sems[NBUF:]
    h = lax.axis_index("c")
    b = (lax.axis_index("s") + 8 * h) % B

    dur_cp = pltpu.async_copy(dur.at[b], dur_v, gsem[0])

    zeros16i = jnp.zeros((LANES,), jnp.int32)
    for i in range(NFRM_CH):
        z_v[pl.ds(i * LANES, LANES)] = zeros16i
    dur_cp.wait()

    top = jnp.full((LANES,), LANES - 1, jnp.int32)

    def _bcast_last(v):
        return v.at[top].get(mode="promise_in_bounds")

    carry = jnp.zeros((LANES,), jnp.int32)
    ids0 = lax.broadcasted_iota(jnp.int32, (LANES,), 0)
    for i in range(NTOK_CH):
        dv = dur_v[pl.ds(i * LANES, LANES)]
        d16 = jnp.maximum(dv, 1)
        d_v[pl.ds(i * LANES, LANES)] = d16
        cum16 = plsc.cumsum(d16) + carry
        starts = cum16 - d16
        carry = _bcast_last(cum16)
        mask = starts < L
        starts_c = jnp.minimum(starts, L - 1)
        plsc.store_scatter(z_v, [starts_c], ids0 + (i * LANES), mask=mask)
    total = jnp.max(carry)

    @pl.when(h == b % 2)
    def _():
        pltpu.sync_copy(d_v, d_out.at[b])

    zeros16f = jnp.zeros((LANES,), jnp.float32)

    def _zero_rows(ref, lo, hi):
        def body(r, _):
            for k in range(D // LANES):
                ref[r, pl.ds(k * LANES, LANES)] = zeros16f
            return 0
        lax.fori_loop(lo, hi, body, 0)


    def _slot(k):
        c0 = CHUNKS_A[k] if k < len(CHUNKS_A) else 0
        c1 = CHUNKS_B[k] if k < len(CHUNKS_B) else 0
        cid = jnp.where(h == 0, c0, c1)
        if k < len(CHUNKS_A) and k < len(CHUNKS_B):
            valid = (h == 0) | (h == 1)
        elif k < len(CHUNKS_A):
            valid = h == 0
        else:
            valid = h == 1
        start = cid * ROWS
        fb = pl.multiple_of(start, ROWS)
        live = jnp.clip(total - start, 0, ROWS)
        return fb, live, valid

    def _issue(k):
        buf = k % NBUF
        fb, live, valid = _slot(k)

        @pl.when(valid & (live > 0))
        def _():
            pltpu.async_copy(table.at[gidx_v.at[pl.ds(fb, ROWS)]],
                             rows_v.at[buf], gsem[buf])

    def _finish(k):
        buf = k % NBUF
        fb, live, valid = _slot(k)

        @pl.when(valid & (live > 0))
        def _():
            pltpu.make_async_copy(table.at[pl.ds(0, ROWS)], rows_v.at[buf],
                                  gsem[buf]).wait()

            @pl.when(live < ROWS)
            def _():
                _zero_rows(rows_v.at[buf], live, ROWS)

            pltpu.async_copy(rows_v.at[buf], out.at[b, pl.ds(fb, ROWS)],
                             wsem[buf])

        @pl.when(valid & (live == 0))
        def _():
            pltpu.async_copy(zero_v, out.at[b, pl.ds(fb, ZROWS)], wsem[buf])

    def _drain_write(k):
        buf = k % NBUF
        _, _, valid = _slot(k)

        @pl.when(valid)
        def _():
            pltpu.make_async_copy(table.at[pl.ds(0, ROWS)], rows_v.at[buf],
                                  wsem[buf]).wait()

    mcarry = jnp.zeros((LANES,), jnp.int32)
    base_row = b * T
    for i in range(NFRM_CH):
        zc = z_v[pl.ds(i * LANES, LANES)]
        m = jnp.maximum(plsc.cummax(zc), mcarry)
        mcarry = _bcast_last(m)
        gidx_v[pl.ds(i * LANES, LANES)] = m + base_row
        if i % 4 == 1 and (i - 1) // 4 < NBUF:
            @pl.when(h == 0)
            def _(k=(i - 1) // 4):
                _issue(k)
        if i % 4 == 3 and (i - 3) // 4 < NBUF:
            @pl.when(h == 1)
            def _(k=(i - 3) // 4):
                _issue(k)

    _zero_rows(zero_v, 0, ZROWS)
    for k in range(NSLOTS):
        _finish(k)
        if k + NBUF < NSLOTS:
            _drain_write(k)
            _issue(k + NBUF)
    for k in range(max(NSLOTS - NBUF, 0), NSLOTS):
        _drain_write(k)


def kernel(sequences, durations, max_mel_length):
    table = sequences.reshape(B * T, D)
    mesh = plsc.VectorSubcoreMesh(core_axis_name="c", subcore_axis_name="s")
    run = functools.partial(
        pl.kernel,
        mesh=mesh,
        compiler_params=pltpu.CompilerParams(needs_layout_passes=False),
        out_type=(jax.ShapeDtypeStruct((B, L, D), jnp.float32),
                  jax.ShapeDtypeStruct((B, T), jnp.int32)),
        scratch_types=[
            pltpu.VMEM((T,), jnp.int32),
            pltpu.VMEM((T,), jnp.int32),
            pltpu.VMEM((L,), jnp.int32),
            pltpu.VMEM((L,), jnp.int32),
            pltpu.VMEM((NBUF, ROWS, D), jnp.float32),
            pltpu.VMEM((ZROWS, D), jnp.float32),
        ] + [pltpu.SemaphoreType.DMA] * (2 * NBUF),
    )(_lr_body)
    out, d = run(table, durations)
    return out, d
